# baseline (speedup 1.0000x reference)
import functools

import jax
import jax.numpy as jnp
from jax import lax
from jax.experimental import pallas as pl
from jax.experimental.pallas import tpu as pltpu
from jax.experimental.pallas import tpu_sc as plsc

BATCH = 16384
VOCAB = 1000000
HIDDEN = 64
NUM_SAMPLE = 5

NPAIR = BATCH * NUM_SAMPLE
NTILE = 16
PT = NPAIR // NTILE
BT = PT // NUM_SAMPLE
NJ = PT // 128
HC = HIDDEN // 2


def _planes_body(inputsT_hbm, idx_hbm, tableT_hbm, tailT_hbm, part_hbm,
                 idx_v, biv, val_v, acc_v, inp_v, tl_v, sp,
                 sem_p, sem_g, sem_i, sem_t):
    c = lax.axis_index("c")
    s = lax.axis_index("s")
    h0 = c * HC
    b0 = s * BT

    pltpu.sync_copy(idx_hbm.at[s], idx_v)

    lane = lax.iota(jnp.int32, 16)

    def init_body(j, _):
        for l in range(8):
            sl = pl.ds(l * 16, 16)
            base = j * 128 + l * 16
            biv[j, sl] = (base + lane) // NUM_SAMPLE
            acc_v[j, sl] = jnp.zeros((16,), jnp.float32)
        return 0

    lax.fori_loop(0, NJ, init_body, 0)

    NCH = 8
    CH = 124928
    TAIL = 640

    def fire_plane(hh):
        for i in range(NCH):
            @pl.when(s == i)
            def _(i=i):
                pltpu.async_copy(
                    tableT_hbm.at[hh].at[pl.ds(i * CH, CH)],
                    sp.at[pl.ds(i * CH, CH)], sem_p)

        @pl.when(s == NCH)
        def _():
            pltpu.async_copy(tailT_hbm.at[hh], tl_v, sem_t)

    def drain_plane():
        for i in range(NCH):
            @pl.when(s == i)
            def _(i=i):
                pltpu.make_async_copy(
                    tableT_hbm.at[h0].at[pl.ds(i * CH, CH)],
                    sp.at[pl.ds(i * CH, CH)], sem_p).wait()

        @pl.when(s == NCH)
        def _():
            pltpu.make_async_copy(tailT_hbm.at[h0], tl_v, sem_t).wait()
            pltpu.sync_copy(tl_v, sp.at[pl.ds(VOCAB - TAIL, TAIL)])

    fire_plane(h0)

    def plane_body(k, _):
        h = h0 + k

        drain_plane()

        plsc.subcore_barrier()

        inp_cp = pltpu.async_copy(
            inputsT_hbm.at[h, pl.ds(b0, BT)], inp_v, sem_i)
        gathers = [
            pltpu.async_copy(sp.at[idx_v.at[j]], val_v.at[j], sem_g)
            for j in range(NJ)
        ]
        inp_cp.wait()
        for g in gathers:
            g.wait()

        plsc.subcore_barrier()

        @pl.when(k < HC - 1)
        def _():
            fire_plane(h + 1)

        def comp(j, _):
            for l in range(8):
                sl = pl.ds(l * 16, 16)
                x = plsc.load_gather(inp_v, [biv[j, sl]])
                acc_v[j, sl] = acc_v[j, sl] + val_v[j, sl] * x
            return 0

        lax.fori_loop(0, NJ, comp, 0)
        return 0

    lax.fori_loop(0, HC, plane_body, 0)

    pltpu.sync_copy(acc_v, part_hbm.at[c, s])


@jax.jit
def _planes(inputsT, idx3, tableT, tailT):
    mesh = plsc.VectorSubcoreMesh(core_axis_name="c", subcore_axis_name="s")
    f = pl.kernel(
        _planes_body,
        mesh=mesh,
        out_type=jax.ShapeDtypeStruct((2, NTILE, NJ, 128), jnp.float32),
        scratch_types=[
            pltpu.VMEM((NJ, 128), jnp.int32),
            pltpu.VMEM((NJ, 128), jnp.int32),
            pltpu.VMEM((NJ, 128), jnp.float32),
            pltpu.VMEM((NJ, 128), jnp.float32),
            pltpu.VMEM((BT,), jnp.float32),
            pltpu.VMEM((640,), jnp.float32),
            pltpu.VMEM_SHARED((VOCAB,), jnp.float32),
            pltpu.SemaphoreType.DMA,
            pltpu.SemaphoreType.DMA,
            pltpu.SemaphoreType.DMA,
            pltpu.SemaphoreType.DMA,
        ],
        compiler_params=pltpu.CompilerParams(needs_layout_passes=False),
    )
    return f(inputsT, idx3, tableT, tailT)


def _combine_body(p_ref, o_ref):
    z = p_ref[0] + p_ref[1]
    o_ref[...] = 1.0 / (1.0 + jnp.exp(-z))


@jax.jit
def _combine(part):
    return pl.pallas_call(
        _combine_body,
        out_shape=jax.ShapeDtypeStruct((NPAIR // 128, 128), jnp.float32),
    )(part)


def kernel(inputs, idxs, out_embedding):
    tableT = out_embedding.T
    inputsT = inputs.T
    tailT = out_embedding[VOCAB - 640:, :].T
    idx3 = idxs.reshape(-1).astype(jnp.int32).reshape(NTILE, NJ, 128)
    part = _planes(inputsT, idx3, tableT, tailT)
    out = _combine(part.reshape(2, NPAIR // 128, 128))
    return out.reshape(BATCH, NUM_SAMPLE)
